# baseline (speedup 1.0000x reference)
import functools

import numpy as np
import jax
import jax.numpy as jnp
from jax import lax
from jax.experimental import pallas as pl
from jax.experimental.pallas import tpu as pltpu
from jax.experimental.pallas import tpu_sc as plsc

N = 10000
E = 160000
G = 64
NR = 32
CUT = 6.0
RMAX = 10.0
HID = 128
NLAYERS = 4
NFC = 4

NP = 10240
EP = 163840
NW = 32
EW = EP // NW
ET = EP // 16
C = 64
NCH = EW // C
NCHS = ET // C
RT = NP // 16

BN = 256
BE = 512
GN = NP // BN
GE = EP // BE

_SQS = float(1.0 / np.sqrt(HID))
_GAMMA = float(1.0 / (2.0 * (CUT / NR) ** 2))
_MUSTEP = float(CUT / (NR - 1))
_NEG = -1e30

_f32 = jnp.float32
_i32 = jnp.int32


def _sc_mesh():
    return plsc.VectorSubcoreMesh(core_axis_name="c", subcore_axis_name="s")


R = 4


def _make_gather1(w):
    @functools.partial(
        pl.kernel, mesh=_sc_mesh(),
        out_type=jax.ShapeDtypeStruct((EP, w), _f32),
        scratch_types=[pltpu.VMEM((NCH, C), _i32),
                       pltpu.VMEM((R, C, w), _f32),
                       pltpu.SemaphoreType.DMA, pltpu.SemaphoreType.DMA],
    )
    def k(t_h, i_h, o_h, idx, buf, gs, ws):
        wid = lax.axis_index("s") * 2 + lax.axis_index("c")
        base = wid * EW
        pltpu.sync_copy(i_h.at[pl.ds(wid * NCH, NCH)], idx)
        for r in range(R - 1):
            pltpu.async_copy(t_h.at[idx.at[r]], buf.at[r], gs)

        def body(j, carry):
            p = lax.rem(j, R)
            pm1 = lax.rem(j + R - 1, R)

            @pl.when(j >= 1)
            def _():
                off1 = base + (j - 1) * C
                pltpu.make_async_copy(
                    buf.at[pm1], o_h.at[pl.ds(off1, C)], ws).wait()

            @pl.when(j + R - 1 < NCH)
            def _():
                pltpu.async_copy(t_h.at[idx.at[j + R - 1]], buf.at[pm1], gs)

            off = base + j * C
            pltpu.make_async_copy(t_h.at[idx.at[j]], buf.at[p], gs).wait()
            pltpu.async_copy(buf.at[p], o_h.at[pl.ds(off, C)], ws)
            return carry

        lax.fori_loop(0, NCH, body, 0)
        pl_ = (NCH - 1) % R
        offl = base + (NCH - 1) * C
        pltpu.make_async_copy(buf.at[pl_], o_h.at[pl.ds(offl, C)], ws).wait()

    return k


def _gather_pos_call(pos128, dsti2, srci2):
    g = _make_gather1(128)
    return g(pos128, dsti2), g(pos128, srci2)


def _gather_layer_call(qqe, kv, dsti2, srci2):
    return _make_gather1(256)(qqe, dsti2), _make_gather1(256)(kv, srci2)


RS = 3


def _scatter_call(p1, p2, dsti2):
    @functools.partial(
        pl.kernel, mesh=_sc_mesh(),
        out_type=[jax.ShapeDtypeStruct((NP, 128), _f32),
                  jax.ShapeDtypeStruct((NP, 128), _f32)],
        scratch_types=[pltpu.VMEM_SHARED((NP, 128), _f32),
                       pltpu.VMEM((RS, C, 128), _f32),
                       pltpu.VMEM((NCHS, C), _i32),
                       pltpu.SemaphoreType.DMA, pltpu.SemaphoreType.DMA],
    )
    def k(p1_h, p2_h, d_h, o1_h, o2_h, acc, buf, idx, lsem, asem):
        cid = lax.axis_index("c")
        sid = lax.axis_index("s")
        pltpu.sync_copy(d_h.at[pl.ds(sid * NCHS, NCHS)], idx)

        def zrow(i, carry):
            for t in range(128 // 16):
                buf[0, i, pl.ds(16 * t, 16)] = jnp.zeros((16,), _f32)
            return carry

        lax.fori_loop(0, C, zrow, 0)

        def zcp(i, carry):
            pltpu.sync_copy(buf.at[0], acc.at[pl.ds(sid * RT + i * C, C)])
            return carry

        lax.fori_loop(0, RT // C, zcp, 0)
        plsc.subcore_barrier()

        def accumulate(src_h):
            for r in range(RS - 1):
                pltpu.async_copy(src_h.at[pl.ds(sid * ET + r * C, C)],
                                 buf.at[r], lsem)

            def body(j, carry):
                p = lax.rem(j, RS)
                pm1 = lax.rem(j + RS - 1, RS)

                @pl.when(j >= 1)
                def _():
                    pltpu.make_async_copy(buf.at[pm1],
                                          acc.at[idx.at[j - 1]], asem).wait()

                @pl.when(j + RS - 1 < NCHS)
                def _():
                    off1 = sid * ET + (j + RS - 1) * C
                    pltpu.async_copy(src_h.at[pl.ds(off1, C)], buf.at[pm1],
                                     lsem)

                off = sid * ET + j * C
                pltpu.make_async_copy(src_h.at[pl.ds(off, C)], buf.at[p],
                                      lsem).wait()
                pltpu.async_copy(buf.at[p], acc.at[idx.at[j]], asem,
                                 add=True)
                return carry

            lax.fori_loop(0, NCHS, body, 0)
            pl_ = (NCHS - 1) % RS
            pltpu.make_async_copy(buf.at[pl_], acc.at[idx.at[NCHS - 1]],
                                  asem).wait()

        @pl.when(cid == 0)
        def _():
            accumulate(p1_h)

        @pl.when(cid == 1)
        def _():
            accumulate(p2_h)

        plsc.subcore_barrier()
        r = sid * RT

        @pl.when(cid == 0)
        def _():
            pltpu.sync_copy(acc.at[pl.ds(r, RT)], o1_h.at[pl.ds(r, RT)])

        @pl.when(cid == 1)
        def _():
            pltpu.sync_copy(acc.at[pl.ds(r, RT)], o2_h.at[pl.ds(r, RT)])

    return k(p1, p2, dsti2)


def _init_call(x256, pos128, batchi, mol_x, wa, wb128, wc, b):
    def body(x_r, p_r, bt_r, mx_r, wa_r, wb_r, wc_r, b_r, o_r):
        mw = jnp.dot(mx_r[...], wc_r[...], preferred_element_type=_f32,
                    precision=lax.Precision.HIGHEST)
        gids = lax.broadcasted_iota(_i32, (1, G), 1)
        oh = (bt_r[...] == gids).astype(_f32)
        h = jnp.dot(x_r[...], wa_r[...], preferred_element_type=_f32,
                    precision=lax.Precision.HIGHEST)
        h += jnp.dot(p_r[...] * (1.0 / RMAX), wb_r[...],
                     preferred_element_type=_f32,
                    precision=lax.Precision.HIGHEST)
        h += jnp.dot(oh, mw, preferred_element_type=_f32,
                    precision=lax.Precision.HIGHEST)
        o_r[...] = jax.nn.gelu(h + b_r[...])

    return pl.pallas_call(
        body,
        grid=(GN,),
        in_specs=[
            pl.BlockSpec((BN, 256), lambda i: (i, 0)),
            pl.BlockSpec((BN, 128), lambda i: (i, 0)),
            pl.BlockSpec((BN, 1), lambda i: (i, 0)),
            pl.BlockSpec((G, NR), lambda i: (0, 0)),
            pl.BlockSpec((256, HID), lambda i: (0, 0)),
            pl.BlockSpec((128, HID), lambda i: (0, 0)),
            pl.BlockSpec((NR, HID), lambda i: (0, 0)),
            pl.BlockSpec((1, HID), lambda i: (0, 0)),
        ],
        out_specs=pl.BlockSpec((BN, HID), lambda i: (i, 0)),
        out_shape=jax.ShapeDtypeStruct((NP, HID), _f32),
    )(x256, pos128, batchi, mol_x, wa, wb128, wc, b)


def _rbf_call(pd, ps):
    def body(pd_r, ps_r, r128_r):
        diff = pd_r[...] - ps_r[...] + 1e-8
        c128 = lax.broadcasted_iota(_i32, (BE, 128), 1)
        d2 = jnp.sum(jnp.where(c128 < 3, diff * diff, 0.0), axis=1,
                     keepdims=True)
        d = jnp.sqrt(d2)
        mu128 = c128.astype(_f32) * _MUSTEP
        vals = jnp.exp(-_GAMMA * (d - mu128) ** 2)
        r128_r[...] = jnp.where(c128 < NR, vals,
                                jnp.where(c128 == NR, 1.0, 0.0))

    return pl.pallas_call(
        body,
        grid=(GE,),
        in_specs=[pl.BlockSpec((BE, 128), lambda i: (i, 0)),
                  pl.BlockSpec((BE, 128), lambda i: (i, 0))],
        out_specs=pl.BlockSpec((BE, 128), lambda i: (i, 0)),
        out_shape=jax.ShapeDtypeStruct((EP, 128), _f32),
    )(pd, ps)


def _qkv_call(h, wq, wk, wv, wekT):
    def body(h_r, wq_r, wk_r, wv_r, we_r, a_r, b_r):
        hb = h_r[...]
        q = jnp.dot(hb, wq_r[...], preferred_element_type=_f32,
                    precision=lax.Precision.HIGHEST)
        qe = jnp.dot(q, we_r[...], preferred_element_type=_f32,
                    precision=lax.Precision.HIGHEST)
        a_r[:, pl.ds(0, 128)] = q
        a_r[:, pl.ds(128, 128)] = jnp.concatenate(
            [qe, jnp.zeros((BN, 128 - NR), _f32)], axis=1)
        b_r[:, pl.ds(0, 128)] = jnp.dot(hb, wk_r[...],
                                        preferred_element_type=_f32,
                    precision=lax.Precision.HIGHEST)
        b_r[:, pl.ds(128, 128)] = jnp.dot(hb, wv_r[...],
                                          preferred_element_type=_f32,
                    precision=lax.Precision.HIGHEST)

    return pl.pallas_call(
        body,
        grid=(GN,),
        in_specs=[pl.BlockSpec((BN, HID), lambda i: (i, 0)),
                  pl.BlockSpec((HID, HID), lambda i: (0, 0)),
                  pl.BlockSpec((HID, HID), lambda i: (0, 0)),
                  pl.BlockSpec((HID, HID), lambda i: (0, 0)),
                  pl.BlockSpec((HID, NR), lambda i: (0, 0))],
        out_specs=[pl.BlockSpec((BN, 256), lambda i: (i, 0)),
                   pl.BlockSpec((BN, 256), lambda i: (i, 0))],
        out_shape=[jax.ShapeDtypeStruct((NP, 256), _f32),
                   jax.ShapeDtypeStruct((NP, 256), _f32)],
    )(h, wq, wk, wv, wekT)


def _logits_call(qqed, kvs, rbf128):
    def body(qd_r, qe_r, ks_r, rb_r, l_r, m_r):
        i = pl.program_id(0)
        cols = lax.broadcasted_iota(_i32, (BE, 128), 1)
        qe = jnp.where(cols < NR, qe_r[...], 0.0)
        rb = jnp.where(cols < NR, rb_r[...], 0.0)
        raw = (jnp.sum(qd_r[...] * ks_r[...], axis=1, keepdims=True)
               + jnp.sum(qe * rb, axis=1, keepdims=True)) * _SQS
        eid = lax.broadcasted_iota(_i32, (BE, 1), 0) + i * BE
        lv = jnp.where(eid < E, raw, _NEG)
        l_r[...] = lv

        @pl.when(i == 0)
        def _():
            m_r[...] = jnp.full((1, 1), _NEG, _f32)

        m_r[...] = jnp.maximum(m_r[...], jnp.max(lv))

    return pl.pallas_call(
        body,
        grid=(GE,),
        in_specs=[pl.BlockSpec((BE, 128), lambda i: (i, 0)),
                  pl.BlockSpec((BE, 128), lambda i: (i, 1)),
                  pl.BlockSpec((BE, 128), lambda i: (i, 0)),
                  pl.BlockSpec((BE, 128), lambda i: (i, 0))],
        out_specs=[pl.BlockSpec((BE, 1), lambda i: (i, 0)),
                   pl.BlockSpec((1, 1), lambda i: (0, 0))],
        out_shape=[jax.ShapeDtypeStruct((EP, 1), _f32),
                   jax.ShapeDtypeStruct((1, 1), _f32)],
    )(qqed, qqed, kvs, rbf128)


def _payload_call(logits, m, kvs, rbf128):
    def body(l_r, m_r, vs_r, rb_r, p1_r, p2_r):
        a = jnp.exp(l_r[...] - m_r[...])
        p1_r[...] = a * vs_r[...]
        p2_r[...] = a * rb_r[...]

    return pl.pallas_call(
        body,
        grid=(GE,),
        in_specs=[pl.BlockSpec((BE, 1), lambda i: (i, 0)),
                  pl.BlockSpec((1, 1), lambda i: (0, 0)),
                  pl.BlockSpec((BE, 128), lambda i: (i, 1)),
                  pl.BlockSpec((BE, 128), lambda i: (i, 0))],
        out_specs=[pl.BlockSpec((BE, 128), lambda i: (i, 0)),
                   pl.BlockSpec((BE, 128), lambda i: (i, 0))],
        out_shape=[jax.ShapeDtypeStruct((EP, 128), _f32),
                   jax.ShapeDtypeStruct((EP, 128), _f32)],
    )(logits, m, kvs, rbf128)


def _epilogue_call(o1, o2, h, wev, wr, lng, lnb):
    def body(a_r, c_r, h_r, we_r, wr_r, g_r, be_r, o_r):
        acc1 = a_r[...]
        acc2 = c_r[...]
        rows = lax.broadcasted_iota(_i32, (128, 1), 0)
        cols = lax.broadcasted_iota(_i32, (128, NR), 1)
        ssel = (rows == cols).astype(_f32)
        dsel = (rows == NR).astype(_f32)
        s = jnp.dot(acc2, ssel, preferred_element_type=_f32,
                    precision=lax.Precision.HIGHEST)
        denom = jnp.dot(acc2, dsel, preferred_element_type=_f32,
                    precision=lax.Precision.HIGHEST)
        agg = (acc1 + jnp.dot(s, we_r[...], preferred_element_type=_f32,
                    precision=lax.Precision.HIGHEST)) \
            / (denom + 1e-16)
        out = agg + jnp.dot(h_r[...], wr_r[...], preferred_element_type=_f32,
                    precision=lax.Precision.HIGHEST)
        mean = jnp.mean(out, axis=1, keepdims=True)
        cen = out - mean
        var = jnp.mean(cen * cen, axis=1, keepdims=True)
        hn = cen / jnp.sqrt(var + 1e-5)
        o_r[...] = jax.nn.gelu(hn * g_r[...] + be_r[...])

    return pl.pallas_call(
        body,
        grid=(GN,),
        in_specs=[pl.BlockSpec((BN, 128), lambda i: (i, 0)),
                  pl.BlockSpec((BN, 128), lambda i: (i, 0)),
                  pl.BlockSpec((BN, HID), lambda i: (i, 0)),
                  pl.BlockSpec((NR, HID), lambda i: (0, 0)),
                  pl.BlockSpec((HID, HID), lambda i: (0, 0)),
                  pl.BlockSpec((1, HID), lambda i: (0, 0)),
                  pl.BlockSpec((1, HID), lambda i: (0, 0))],
        out_specs=pl.BlockSpec((BN, HID), lambda i: (i, 0)),
        out_shape=jax.ShapeDtypeStruct((NP, HID), _f32),
    )(o1, o2, h, wev, wr, lng, lnb)


def _emb_pool_call(h, batchi, wemb, bemb):
    def body(h_r, bt_r, w_r, b_r, o_r):
        i = pl.program_id(0)
        g = jax.nn.gelu(jnp.dot(h_r[...], w_r[...],
                                preferred_element_type=_f32,
                    precision=lax.Precision.HIGHEST) + b_r[...])

        @pl.when(i == 0)
        def _():
            o_r[...] = jnp.full((G, 2 * HID), _NEG, _f32)

        lo = bt_r[0, 0]
        hi = jnp.minimum(bt_r[BN - 1, 0], G - 1) + 1

        def gbody(gi, carry):
            mask = bt_r[...] == gi
            vals = jnp.where(mask, g, _NEG)
            m = jnp.max(vals, axis=0, keepdims=True)
            cur = o_r[pl.ds(gi, 1), :]
            o_r[pl.ds(gi, 1), :] = jnp.maximum(cur, m)
            return carry

        lax.fori_loop(lo, hi, gbody, 0)

    return pl.pallas_call(
        body,
        grid=(GN,),
        in_specs=[pl.BlockSpec((BN, HID), lambda i: (i, 0)),
                  pl.BlockSpec((BN, 1), lambda i: (i, 0)),
                  pl.BlockSpec((HID, 2 * HID), lambda i: (0, 0)),
                  pl.BlockSpec((1, 2 * HID), lambda i: (0, 0))],
        out_specs=pl.BlockSpec((G, 2 * HID), lambda i: (0, 0)),
        out_shape=jax.ShapeDtypeStruct((G, 2 * HID), _f32),
    )(h, batchi, wemb, bemb)


def _head_call(pooled, wfcs, bfcs, wout, bout):
    def body(p_r, w0, w1, w2, w3, b0, b1, b2, b3, wo, bo, o_r):
        f = p_r[...]
        f = jnp.where(f > -1e29, f, 0.0)
        for w_r, b_r in ((w0, b0), (w1, b1), (w2, b2), (w3, b3)):
            f = jax.nn.gelu(jnp.dot(f, w_r[...],
                                    preferred_element_type=_f32,
                    precision=lax.Precision.HIGHEST) + b_r[...])
        o_r[...] = jnp.dot(f, wo[...], preferred_element_type=_f32,
                    precision=lax.Precision.HIGHEST) + bo[...]

    emb = 2 * HID
    return pl.pallas_call(
        body,
        in_specs=[pl.BlockSpec((G, emb), lambda: (0, 0))]
        + [pl.BlockSpec((emb, emb), lambda: (0, 0))] * 4
        + [pl.BlockSpec((1, emb), lambda: (0, 0))] * 4
        + [pl.BlockSpec((emb, 12), lambda: (0, 0)),
           pl.BlockSpec((1, 12), lambda: (0, 0))],
        out_specs=pl.BlockSpec((G, 12), lambda: (0, 0)),
        out_shape=jax.ShapeDtypeStruct((G, 12), _f32),
    )(pooled, *wfcs, *bfcs, wout, bout)


def kernel(x, pos, edge_index, batch, mol_x, params):
    srci = jnp.zeros((EP,), _i32).at[:E].set(
        edge_index[0].astype(_i32)).reshape(EP // C, C)
    dsti = jnp.zeros((EP,), _i32).at[:E].set(
        edge_index[1].astype(_i32)).reshape(EP // C, C)
    pos128 = jnp.zeros((NP, 128), _f32).at[:N, :3].set(pos)
    x256 = jnp.zeros((NP, 256), _f32).at[:N].set(x[:, :256])
    batchi = jnp.full((NP, 1), G, _i32).at[:N, 0].set(batch.astype(_i32))

    wi = params['W_init']
    wa = wi[:256]
    wb128 = jnp.zeros((128, HID), _f32).at[:3].set(wi[256:259])
    wc = wi[259:291]
    b_init = params['b_init'].reshape(1, HID)

    h = _init_call(x256, pos128, batchi, mol_x, wa, wb128, wc, b_init)
    pd, ps = _gather_pos_call(pos128, dsti, srci)
    rbf128 = _rbf_call(pd, ps)

    for l in range(NLAYERS):
        wekT = params['Wek%d' % l].T
        qqe, kv = _qkv_call(h, params['Wq%d' % l], params['Wk%d' % l],
                            params['Wv%d' % l], wekT)
        qqed, kvs = _gather_layer_call(qqe, kv, dsti, srci)
        logits, m = _logits_call(qqed, kvs, rbf128)
        p1, p2 = _payload_call(logits, m, kvs, rbf128)
        o1, o2 = _scatter_call(p1, p2, dsti)
        h = _epilogue_call(o1, o2, h,
                           params['Wev%d' % l], params['Wr%d' % l],
                           params['lng%d' % l].reshape(1, HID),
                           params['lnb%d' % l].reshape(1, HID))

    pooled = _emb_pool_call(h, batchi, params['W_emb'],
                            params['b_emb'].reshape(1, 2 * HID))
    out = _head_call(pooled,
                     [params['Wfc%d' % l] for l in range(NFC)],
                     [params['bfc%d' % l].reshape(1, 2 * HID)
                      for l in range(NFC)],
                     params['W_out'], params['b_out'].reshape(1, 12))
    return out


# speedup vs baseline: 1.1680x; 1.1680x over previous
import functools

import numpy as np
import jax
import jax.numpy as jnp
from jax import lax
from jax.experimental import pallas as pl
from jax.experimental.pallas import tpu as pltpu
from jax.experimental.pallas import tpu_sc as plsc

N = 10000
E = 160000
G = 64
NR = 32
CUT = 6.0
RMAX = 10.0
HID = 128
NLAYERS = 4
NFC = 4

NP = 10240
EP = 163840
NW = 32
EW = EP // NW
ET = EP // 16
C = 64
NCH = EW // C
NCHS = ET // C
RT = NP // 16

BN = 256
BE = 512
GN = NP // BN
GE = EP // BE

_SQS = float(1.0 / np.sqrt(HID))
_GAMMA = float(1.0 / (2.0 * (CUT / NR) ** 2))
_MUSTEP = float(CUT / (NR - 1))
_NEG = -1e30

_f32 = jnp.float32
_i32 = jnp.int32


def _sc_mesh():
    return plsc.VectorSubcoreMesh(core_axis_name="c", subcore_axis_name="s")


R = 3


def _make_gather2(w):
    @functools.partial(
        pl.kernel, mesh=_sc_mesh(),
        out_type=[jax.ShapeDtypeStruct((EP, w), _f32),
                  jax.ShapeDtypeStruct((EP, w), _f32)],
        scratch_types=[pltpu.VMEM((NCH, C), _i32), pltpu.VMEM((NCH, C), _i32),
                       pltpu.VMEM((R, C, w), _f32), pltpu.VMEM((R, C, w), _f32),
                       pltpu.SemaphoreType.DMA, pltpu.SemaphoreType.DMA,
                       pltpu.SemaphoreType.DMA, pltpu.SemaphoreType.DMA],
    )
    def k(a_h, b_h, d_h, s_h, ao_h, bo_h, idxd, idxs, bufa, bufb,
          gsa, gsb, wsa, wsb):
        wid = lax.axis_index("s") * 2 + lax.axis_index("c")
        base = wid * EW
        pltpu.sync_copy(d_h.at[pl.ds(wid * NCH, NCH)], idxd)
        pltpu.sync_copy(s_h.at[pl.ds(wid * NCH, NCH)], idxs)
        for r in range(R - 1):
            pltpu.async_copy(a_h.at[idxd.at[r]], bufa.at[r], gsa)
            pltpu.async_copy(b_h.at[idxs.at[r]], bufb.at[r], gsb)

        def body(j, carry):
            p = lax.rem(j, R)
            pm1 = lax.rem(j + R - 1, R)

            @pl.when(j >= 1)
            def _():
                off1 = base + (j - 1) * C
                pltpu.make_async_copy(
                    bufa.at[pm1], ao_h.at[pl.ds(off1, C)], wsa).wait()
                pltpu.make_async_copy(
                    bufb.at[pm1], bo_h.at[pl.ds(off1, C)], wsb).wait()

            @pl.when(j + R - 1 < NCH)
            def _():
                pltpu.async_copy(a_h.at[idxd.at[j + R - 1]], bufa.at[pm1],
                                 gsa)
                pltpu.async_copy(b_h.at[idxs.at[j + R - 1]], bufb.at[pm1],
                                 gsb)

            off = base + j * C
            pltpu.make_async_copy(a_h.at[idxd.at[j]], bufa.at[p], gsa).wait()
            pltpu.async_copy(bufa.at[p], ao_h.at[pl.ds(off, C)], wsa)
            pltpu.make_async_copy(b_h.at[idxs.at[j]], bufb.at[p], gsb).wait()
            pltpu.async_copy(bufb.at[p], bo_h.at[pl.ds(off, C)], wsb)
            return carry

        lax.fori_loop(0, NCH, body, 0)
        pl_ = (NCH - 1) % R
        offl = base + (NCH - 1) * C
        pltpu.make_async_copy(bufa.at[pl_], ao_h.at[pl.ds(offl, C)],
                              wsa).wait()
        pltpu.make_async_copy(bufb.at[pl_], bo_h.at[pl.ds(offl, C)],
                              wsb).wait()

    return k


def _gather_pos_call(pos128, dsti2, srci2):
    return _make_gather2(128)(pos128, pos128, dsti2, srci2)


def _gather_layer_call(qqe, kv, dsti2, srci2):
    return _make_gather2(256)(qqe, kv, dsti2, srci2)


RS = 3


def _scatter_call(p1, p2, dsti2):
    @functools.partial(
        pl.kernel, mesh=_sc_mesh(),
        out_type=[jax.ShapeDtypeStruct((NP, 128), _f32),
                  jax.ShapeDtypeStruct((NP, 128), _f32)],
        scratch_types=[pltpu.VMEM_SHARED((NP, 128), _f32),
                       pltpu.VMEM((RS, C, 128), _f32),
                       pltpu.VMEM((NCHS, C), _i32),
                       pltpu.SemaphoreType.DMA, pltpu.SemaphoreType.DMA],
    )
    def k(p1_h, p2_h, d_h, o1_h, o2_h, acc, buf, idx, lsem, asem):
        cid = lax.axis_index("c")
        sid = lax.axis_index("s")
        pltpu.sync_copy(d_h.at[pl.ds(sid * NCHS, NCHS)], idx)

        def zrow(i, carry):
            for t in range(128 // 16):
                buf[0, i, pl.ds(16 * t, 16)] = jnp.zeros((16,), _f32)
            return carry

        lax.fori_loop(0, C, zrow, 0)

        def zcp(i, carry):
            pltpu.sync_copy(buf.at[0], acc.at[pl.ds(sid * RT + i * C, C)])
            return carry

        lax.fori_loop(0, RT // C, zcp, 0)
        plsc.subcore_barrier()

        def accumulate(src_h):
            for r in range(RS - 1):
                pltpu.async_copy(src_h.at[pl.ds(sid * ET + r * C, C)],
                                 buf.at[r], lsem)

            def body(j, carry):
                p = lax.rem(j, RS)
                pm1 = lax.rem(j + RS - 1, RS)

                @pl.when(j >= 1)
                def _():
                    pltpu.make_async_copy(buf.at[pm1],
                                          acc.at[idx.at[j - 1]], asem).wait()

                @pl.when(j + RS - 1 < NCHS)
                def _():
                    off1 = sid * ET + (j + RS - 1) * C
                    pltpu.async_copy(src_h.at[pl.ds(off1, C)], buf.at[pm1],
                                     lsem)

                off = sid * ET + j * C
                pltpu.make_async_copy(src_h.at[pl.ds(off, C)], buf.at[p],
                                      lsem).wait()
                pltpu.async_copy(buf.at[p], acc.at[idx.at[j]], asem,
                                 add=True)
                return carry

            lax.fori_loop(0, NCHS, body, 0)
            pl_ = (NCHS - 1) % RS
            pltpu.make_async_copy(buf.at[pl_], acc.at[idx.at[NCHS - 1]],
                                  asem).wait()

        @pl.when(cid == 0)
        def _():
            accumulate(p1_h)

        @pl.when(cid == 1)
        def _():
            accumulate(p2_h)

        plsc.subcore_barrier()
        r = sid * RT

        @pl.when(cid == 0)
        def _():
            pltpu.sync_copy(acc.at[pl.ds(r, RT)], o1_h.at[pl.ds(r, RT)])

        @pl.when(cid == 1)
        def _():
            pltpu.sync_copy(acc.at[pl.ds(r, RT)], o2_h.at[pl.ds(r, RT)])

    return k(p1, p2, dsti2)


def _init_call(x256, pos128, batchi, mol_x, wa, wb128, wc, b):
    def body(x_r, p_r, bt_r, mx_r, wa_r, wb_r, wc_r, b_r, o_r):
        mw = jnp.dot(mx_r[...], wc_r[...], preferred_element_type=_f32,
                    precision=lax.Precision.HIGHEST)
        gids = lax.broadcasted_iota(_i32, (1, G), 1)
        oh = (bt_r[...] == gids).astype(_f32)
        h = jnp.dot(x_r[...], wa_r[...], preferred_element_type=_f32,
                    precision=lax.Precision.HIGHEST)
        h += jnp.dot(p_r[...] * (1.0 / RMAX), wb_r[...],
                     preferred_element_type=_f32,
                    precision=lax.Precision.HIGHEST)
        h += jnp.dot(oh, mw, preferred_element_type=_f32,
                    precision=lax.Precision.HIGHEST)
        o_r[...] = jax.nn.gelu(h + b_r[...])

    return pl.pallas_call(
        body,
        grid=(GN,),
        in_specs=[
            pl.BlockSpec((BN, 256), lambda i: (i, 0)),
            pl.BlockSpec((BN, 128), lambda i: (i, 0)),
            pl.BlockSpec((BN, 1), lambda i: (i, 0)),
            pl.BlockSpec((G, NR), lambda i: (0, 0)),
            pl.BlockSpec((256, HID), lambda i: (0, 0)),
            pl.BlockSpec((128, HID), lambda i: (0, 0)),
            pl.BlockSpec((NR, HID), lambda i: (0, 0)),
            pl.BlockSpec((1, HID), lambda i: (0, 0)),
        ],
        out_specs=pl.BlockSpec((BN, HID), lambda i: (i, 0)),
        out_shape=jax.ShapeDtypeStruct((NP, HID), _f32),
    )(x256, pos128, batchi, mol_x, wa, wb128, wc, b)


def _rbf_call(pd, ps):
    def body(pd_r, ps_r, r128_r):
        diff = pd_r[...] - ps_r[...] + 1e-8
        c128 = lax.broadcasted_iota(_i32, (BE, 128), 1)
        d2 = jnp.sum(jnp.where(c128 < 3, diff * diff, 0.0), axis=1,
                     keepdims=True)
        d = jnp.sqrt(d2)
        mu128 = c128.astype(_f32) * _MUSTEP
        vals = jnp.exp(-_GAMMA * (d - mu128) ** 2)
        r128_r[...] = jnp.where(c128 < NR, vals,
                                jnp.where(c128 == NR, 1.0, 0.0))

    return pl.pallas_call(
        body,
        grid=(GE,),
        in_specs=[pl.BlockSpec((BE, 128), lambda i: (i, 0)),
                  pl.BlockSpec((BE, 128), lambda i: (i, 0))],
        out_specs=pl.BlockSpec((BE, 128), lambda i: (i, 0)),
        out_shape=jax.ShapeDtypeStruct((EP, 128), _f32),
    )(pd, ps)


def _qkv_call(h, wq, wk, wv, wekT):
    def body(h_r, wq_r, wk_r, wv_r, we_r, a_r, b_r):
        hb = h_r[...]
        q = jnp.dot(hb, wq_r[...], preferred_element_type=_f32,
                    precision=lax.Precision.HIGHEST)
        qe = jnp.dot(q, we_r[...], preferred_element_type=_f32,
                    precision=lax.Precision.HIGHEST)
        a_r[:, pl.ds(0, 128)] = q
        a_r[:, pl.ds(128, 128)] = jnp.concatenate(
            [qe, jnp.zeros((BN, 128 - NR), _f32)], axis=1)
        b_r[:, pl.ds(0, 128)] = jnp.dot(hb, wk_r[...],
                                        preferred_element_type=_f32,
                    precision=lax.Precision.HIGHEST)
        b_r[:, pl.ds(128, 128)] = jnp.dot(hb, wv_r[...],
                                          preferred_element_type=_f32,
                    precision=lax.Precision.HIGHEST)

    return pl.pallas_call(
        body,
        grid=(GN,),
        in_specs=[pl.BlockSpec((BN, HID), lambda i: (i, 0)),
                  pl.BlockSpec((HID, HID), lambda i: (0, 0)),
                  pl.BlockSpec((HID, HID), lambda i: (0, 0)),
                  pl.BlockSpec((HID, HID), lambda i: (0, 0)),
                  pl.BlockSpec((HID, NR), lambda i: (0, 0))],
        out_specs=[pl.BlockSpec((BN, 256), lambda i: (i, 0)),
                   pl.BlockSpec((BN, 256), lambda i: (i, 0))],
        out_shape=[jax.ShapeDtypeStruct((NP, 256), _f32),
                   jax.ShapeDtypeStruct((NP, 256), _f32)],
    )(h, wq, wk, wv, wekT)


def _logits_call(qqed, kvs, rbf128):
    def body(qd_r, qe_r, ks_r, rb_r, l_r, m_r):
        i = pl.program_id(0)
        cols = lax.broadcasted_iota(_i32, (BE, 128), 1)
        qe = jnp.where(cols < NR, qe_r[...], 0.0)
        rb = jnp.where(cols < NR, rb_r[...], 0.0)
        raw = (jnp.sum(qd_r[...] * ks_r[...], axis=1, keepdims=True)
               + jnp.sum(qe * rb, axis=1, keepdims=True)) * _SQS
        eid = lax.broadcasted_iota(_i32, (BE, 1), 0) + i * BE
        lv = jnp.where(eid < E, raw, _NEG)
        l_r[...] = lv

        @pl.when(i == 0)
        def _():
            m_r[...] = jnp.full((1, 1), _NEG, _f32)

        m_r[...] = jnp.maximum(m_r[...], jnp.max(lv))

    return pl.pallas_call(
        body,
        grid=(GE,),
        in_specs=[pl.BlockSpec((BE, 128), lambda i: (i, 0)),
                  pl.BlockSpec((BE, 128), lambda i: (i, 1)),
                  pl.BlockSpec((BE, 128), lambda i: (i, 0)),
                  pl.BlockSpec((BE, 128), lambda i: (i, 0))],
        out_specs=[pl.BlockSpec((BE, 1), lambda i: (i, 0)),
                   pl.BlockSpec((1, 1), lambda i: (0, 0))],
        out_shape=[jax.ShapeDtypeStruct((EP, 1), _f32),
                   jax.ShapeDtypeStruct((1, 1), _f32)],
    )(qqed, qqed, kvs, rbf128)


def _payload_call(logits, m, kvs, rbf128):
    def body(l_r, m_r, vs_r, rb_r, p1_r, p2_r):
        a = jnp.exp(l_r[...] - m_r[...])
        p1_r[...] = a * vs_r[...]
        p2_r[...] = a * rb_r[...]

    return pl.pallas_call(
        body,
        grid=(GE,),
        in_specs=[pl.BlockSpec((BE, 1), lambda i: (i, 0)),
                  pl.BlockSpec((1, 1), lambda i: (0, 0)),
                  pl.BlockSpec((BE, 128), lambda i: (i, 1)),
                  pl.BlockSpec((BE, 128), lambda i: (i, 0))],
        out_specs=[pl.BlockSpec((BE, 128), lambda i: (i, 0)),
                   pl.BlockSpec((BE, 128), lambda i: (i, 0))],
        out_shape=[jax.ShapeDtypeStruct((EP, 128), _f32),
                   jax.ShapeDtypeStruct((EP, 128), _f32)],
    )(logits, m, kvs, rbf128)


def _epilogue_call(o1, o2, h, wev, wr, lng, lnb):
    def body(a_r, c_r, h_r, we_r, wr_r, g_r, be_r, o_r):
        acc1 = a_r[...]
        acc2 = c_r[...]
        rows = lax.broadcasted_iota(_i32, (128, 1), 0)
        cols = lax.broadcasted_iota(_i32, (128, NR), 1)
        ssel = (rows == cols).astype(_f32)
        dsel = (rows == NR).astype(_f32)
        s = jnp.dot(acc2, ssel, preferred_element_type=_f32,
                    precision=lax.Precision.HIGHEST)
        denom = jnp.dot(acc2, dsel, preferred_element_type=_f32,
                    precision=lax.Precision.HIGHEST)
        agg = (acc1 + jnp.dot(s, we_r[...], preferred_element_type=_f32,
                    precision=lax.Precision.HIGHEST)) \
            / (denom + 1e-16)
        out = agg + jnp.dot(h_r[...], wr_r[...], preferred_element_type=_f32,
                    precision=lax.Precision.HIGHEST)
        mean = jnp.mean(out, axis=1, keepdims=True)
        cen = out - mean
        var = jnp.mean(cen * cen, axis=1, keepdims=True)
        hn = cen / jnp.sqrt(var + 1e-5)
        o_r[...] = jax.nn.gelu(hn * g_r[...] + be_r[...])

    return pl.pallas_call(
        body,
        grid=(GN,),
        in_specs=[pl.BlockSpec((BN, 128), lambda i: (i, 0)),
                  pl.BlockSpec((BN, 128), lambda i: (i, 0)),
                  pl.BlockSpec((BN, HID), lambda i: (i, 0)),
                  pl.BlockSpec((NR, HID), lambda i: (0, 0)),
                  pl.BlockSpec((HID, HID), lambda i: (0, 0)),
                  pl.BlockSpec((1, HID), lambda i: (0, 0)),
                  pl.BlockSpec((1, HID), lambda i: (0, 0))],
        out_specs=pl.BlockSpec((BN, HID), lambda i: (i, 0)),
        out_shape=jax.ShapeDtypeStruct((NP, HID), _f32),
    )(o1, o2, h, wev, wr, lng, lnb)


def _emb_pool_call(h, batchi, wemb, bemb):
    def body(h_r, bt_r, w_r, b_r, o_r):
        i = pl.program_id(0)
        g = jax.nn.gelu(jnp.dot(h_r[...], w_r[...],
                                preferred_element_type=_f32,
                    precision=lax.Precision.HIGHEST) + b_r[...])

        @pl.when(i == 0)
        def _():
            o_r[...] = jnp.full((G, 2 * HID), _NEG, _f32)

        lo = bt_r[0, 0]
        hi = jnp.minimum(bt_r[BN - 1, 0], G - 1) + 1

        def gbody(gi, carry):
            mask = bt_r[...] == gi
            vals = jnp.where(mask, g, _NEG)
            m = jnp.max(vals, axis=0, keepdims=True)
            cur = o_r[pl.ds(gi, 1), :]
            o_r[pl.ds(gi, 1), :] = jnp.maximum(cur, m)
            return carry

        lax.fori_loop(lo, hi, gbody, 0)

    return pl.pallas_call(
        body,
        grid=(GN,),
        in_specs=[pl.BlockSpec((BN, HID), lambda i: (i, 0)),
                  pl.BlockSpec((BN, 1), lambda i: (i, 0)),
                  pl.BlockSpec((HID, 2 * HID), lambda i: (0, 0)),
                  pl.BlockSpec((1, 2 * HID), lambda i: (0, 0))],
        out_specs=pl.BlockSpec((G, 2 * HID), lambda i: (0, 0)),
        out_shape=jax.ShapeDtypeStruct((G, 2 * HID), _f32),
    )(h, batchi, wemb, bemb)


def _head_call(pooled, wfcs, bfcs, wout, bout):
    def body(p_r, w0, w1, w2, w3, b0, b1, b2, b3, wo, bo, o_r):
        f = p_r[...]
        f = jnp.where(f > -1e29, f, 0.0)
        for w_r, b_r in ((w0, b0), (w1, b1), (w2, b2), (w3, b3)):
            f = jax.nn.gelu(jnp.dot(f, w_r[...],
                                    preferred_element_type=_f32,
                    precision=lax.Precision.HIGHEST) + b_r[...])
        o_r[...] = jnp.dot(f, wo[...], preferred_element_type=_f32,
                    precision=lax.Precision.HIGHEST) + bo[...]

    emb = 2 * HID
    return pl.pallas_call(
        body,
        in_specs=[pl.BlockSpec((G, emb), lambda: (0, 0))]
        + [pl.BlockSpec((emb, emb), lambda: (0, 0))] * 4
        + [pl.BlockSpec((1, emb), lambda: (0, 0))] * 4
        + [pl.BlockSpec((emb, 12), lambda: (0, 0)),
           pl.BlockSpec((1, 12), lambda: (0, 0))],
        out_specs=pl.BlockSpec((G, 12), lambda: (0, 0)),
        out_shape=jax.ShapeDtypeStruct((G, 12), _f32),
    )(pooled, *wfcs, *bfcs, wout, bout)


def kernel(x, pos, edge_index, batch, mol_x, params):
    srci = jnp.zeros((EP,), _i32).at[:E].set(
        edge_index[0].astype(_i32)).reshape(EP // C, C)
    dsti = jnp.zeros((EP,), _i32).at[:E].set(
        edge_index[1].astype(_i32)).reshape(EP // C, C)
    pos128 = jnp.zeros((NP, 128), _f32).at[:N, :3].set(pos)
    x256 = jnp.zeros((NP, 256), _f32).at[:N].set(x[:, :256])
    batchi = jnp.full((NP, 1), G, _i32).at[:N, 0].set(batch.astype(_i32))

    wi = params['W_init']
    wa = wi[:256]
    wb128 = jnp.zeros((128, HID), _f32).at[:3].set(wi[256:259])
    wc = wi[259:291]
    b_init = params['b_init'].reshape(1, HID)

    h = _init_call(x256, pos128, batchi, mol_x, wa, wb128, wc, b_init)
    pd, ps = _gather_pos_call(pos128, dsti, srci)
    rbf128 = _rbf_call(pd, ps)

    for l in range(NLAYERS):
        wekT = params['Wek%d' % l].T
        qqe, kv = _qkv_call(h, params['Wq%d' % l], params['Wk%d' % l],
                            params['Wv%d' % l], wekT)
        qqed, kvs = _gather_layer_call(qqe, kv, dsti, srci)
        logits, m = _logits_call(qqed, kvs, rbf128)
        p1, p2 = _payload_call(logits, m, kvs, rbf128)
        o1, o2 = _scatter_call(p1, p2, dsti)
        h = _epilogue_call(o1, o2, h,
                           params['Wev%d' % l], params['Wr%d' % l],
                           params['lng%d' % l].reshape(1, HID),
                           params['lnb%d' % l].reshape(1, HID))

    pooled = _emb_pool_call(h, batchi, params['W_emb'],
                            params['b_emb'].reshape(1, 2 * HID))
    out = _head_call(pooled,
                     [params['Wfc%d' % l] for l in range(NFC)],
                     [params['bfc%d' % l].reshape(1, 2 * HID)
                      for l in range(NFC)],
                     params['W_out'], params['b_out'].reshape(1, 12))
    return out


# speedup vs baseline: 1.1928x; 1.0212x over previous
import functools

import numpy as np
import jax
import jax.numpy as jnp
from jax import lax
from jax.experimental import pallas as pl
from jax.experimental.pallas import tpu as pltpu
from jax.experimental.pallas import tpu_sc as plsc

N = 10000
E = 160000
G = 64
NR = 32
CUT = 6.0
RMAX = 10.0
HID = 128
NLAYERS = 4
NFC = 4

NP = 10240
EP = 163840
NW = 32
EW = EP // NW
ET = EP // 16
C = 64
NCH = EW // C
NCHS = ET // C
RT = NP // 16

BN = 256
BE = 512
GN = NP // BN
GE = EP // BE

_SQS = float(1.0 / np.sqrt(HID))
_GAMMA = float(1.0 / (2.0 * (CUT / NR) ** 2))
_MUSTEP = float(CUT / (NR - 1))
_NEG = -1e30

_f32 = jnp.float32
_i32 = jnp.int32


def _sc_mesh():
    return plsc.VectorSubcoreMesh(core_axis_name="c", subcore_axis_name="s")


R = 3


def _make_gather2(w):
    @functools.partial(
        pl.kernel, mesh=_sc_mesh(),
        out_type=[jax.ShapeDtypeStruct((EP, w), _f32),
                  jax.ShapeDtypeStruct((EP, w), _f32)],
        scratch_types=[pltpu.VMEM((NCH, C), _i32), pltpu.VMEM((NCH, C), _i32),
                       pltpu.VMEM((R, C, w), _f32), pltpu.VMEM((R, C, w), _f32),
                       pltpu.SemaphoreType.DMA, pltpu.SemaphoreType.DMA,
                       pltpu.SemaphoreType.DMA, pltpu.SemaphoreType.DMA],
    )
    def k(a_h, b_h, d_h, s_h, ao_h, bo_h, idxd, idxs, bufa, bufb,
          gsa, gsb, wsa, wsb):
        wid = lax.axis_index("s") * 2 + lax.axis_index("c")
        base = wid * EW
        pltpu.sync_copy(d_h.at[pl.ds(wid * NCH, NCH)], idxd)
        pltpu.sync_copy(s_h.at[pl.ds(wid * NCH, NCH)], idxs)
        for r in range(R - 1):
            pltpu.async_copy(a_h.at[idxd.at[r]], bufa.at[r], gsa)
            pltpu.async_copy(b_h.at[idxs.at[r]], bufb.at[r], gsb)

        def body(j, carry):
            p = lax.rem(j, R)
            pm1 = lax.rem(j + R - 1, R)

            @pl.when(j >= 1)
            def _():
                off1 = base + (j - 1) * C
                pltpu.make_async_copy(
                    bufa.at[pm1], ao_h.at[pl.ds(off1, C)], wsa).wait()
                pltpu.make_async_copy(
                    bufb.at[pm1], bo_h.at[pl.ds(off1, C)], wsb).wait()

            @pl.when(j + R - 1 < NCH)
            def _():
                pltpu.async_copy(a_h.at[idxd.at[j + R - 1]], bufa.at[pm1],
                                 gsa)
                pltpu.async_copy(b_h.at[idxs.at[j + R - 1]], bufb.at[pm1],
                                 gsb)

            off = base + j * C
            pltpu.make_async_copy(a_h.at[idxd.at[j]], bufa.at[p], gsa).wait()
            pltpu.async_copy(bufa.at[p], ao_h.at[pl.ds(off, C)], wsa)
            pltpu.make_async_copy(b_h.at[idxs.at[j]], bufb.at[p], gsb).wait()
            pltpu.async_copy(bufb.at[p], bo_h.at[pl.ds(off, C)], wsb)
            return carry

        lax.fori_loop(0, NCH, body, 0)
        pl_ = (NCH - 1) % R
        offl = base + (NCH - 1) * C
        pltpu.make_async_copy(bufa.at[pl_], ao_h.at[pl.ds(offl, C)],
                              wsa).wait()
        pltpu.make_async_copy(bufb.at[pl_], bo_h.at[pl.ds(offl, C)],
                              wsb).wait()

    return k


def _gather_pos_call(pos128, dsti2, srci2):
    return _make_gather2(128)(pos128, pos128, dsti2, srci2)


def _gather_layer_call(qqe, kv, dsti2, srci2):
    return _make_gather2(256)(qqe, kv, dsti2, srci2)


RS = 3


def _scatter_call(p1, p2, dsti2):
    @functools.partial(
        pl.kernel, mesh=_sc_mesh(),
        out_type=[jax.ShapeDtypeStruct((NP, 128), _f32),
                  jax.ShapeDtypeStruct((NP, 128), _f32)],
        scratch_types=[pltpu.VMEM_SHARED((NP, 128), _f32),
                       pltpu.VMEM((RS, C, 128), _f32),
                       pltpu.VMEM((NCHS, C), _i32),
                       pltpu.SemaphoreType.DMA, pltpu.SemaphoreType.DMA],
    )
    def k(p1_h, p2_h, d_h, o1_h, o2_h, acc, buf, idx, lsem, asem):
        cid = lax.axis_index("c")
        sid = lax.axis_index("s")
        pltpu.sync_copy(d_h.at[pl.ds(sid * NCHS, NCHS)], idx)

        def zrow(i, carry):
            for t in range(128 // 16):
                buf[0, i, pl.ds(16 * t, 16)] = jnp.zeros((16,), _f32)
            return carry

        lax.fori_loop(0, C, zrow, 0)

        def zcp(i, carry):
            pltpu.sync_copy(buf.at[0], acc.at[pl.ds(sid * RT + i * C, C)])
            return carry

        lax.fori_loop(0, RT // C, zcp, 0)
        plsc.subcore_barrier()

        def accumulate(src_h):
            for r in range(RS - 1):
                pltpu.async_copy(src_h.at[pl.ds(sid * ET + r * C, C)],
                                 buf.at[r], lsem)

            def body(j, carry):
                p = lax.rem(j, RS)
                pm1 = lax.rem(j + RS - 1, RS)

                @pl.when(j >= 1)
                def _():
                    pltpu.make_async_copy(buf.at[pm1],
                                          acc.at[idx.at[j - 1]], asem).wait()

                @pl.when(j + RS - 1 < NCHS)
                def _():
                    off1 = sid * ET + (j + RS - 1) * C
                    pltpu.async_copy(src_h.at[pl.ds(off1, C)], buf.at[pm1],
                                     lsem)

                off = sid * ET + j * C
                pltpu.make_async_copy(src_h.at[pl.ds(off, C)], buf.at[p],
                                      lsem).wait()
                pltpu.async_copy(buf.at[p], acc.at[idx.at[j]], asem,
                                 add=True)
                return carry

            lax.fori_loop(0, NCHS, body, 0)
            pl_ = (NCHS - 1) % RS
            pltpu.make_async_copy(buf.at[pl_], acc.at[idx.at[NCHS - 1]],
                                  asem).wait()

        @pl.when(cid == 0)
        def _():
            accumulate(p1_h)

        @pl.when(cid == 1)
        def _():
            accumulate(p2_h)

        plsc.subcore_barrier()
        r = sid * RT

        @pl.when(cid == 0)
        def _():
            pltpu.sync_copy(acc.at[pl.ds(r, RT)], o1_h.at[pl.ds(r, RT)])

        @pl.when(cid == 1)
        def _():
            pltpu.sync_copy(acc.at[pl.ds(r, RT)], o2_h.at[pl.ds(r, RT)])

    return k(p1, p2, dsti2)


def _dot(a, b):
    return jnp.dot(a, b, preferred_element_type=_f32,
                   precision=lax.Precision.HIGHEST)


def _emit_qkv(h, wq_r, wk_r, wv_r, we_r, a_r, b_r):
    q = _dot(h, wq_r[...])
    qe = _dot(q, we_r[...])
    a_r[:, pl.ds(0, 128)] = q
    a_r[:, pl.ds(128, 128)] = jnp.concatenate(
        [qe, jnp.zeros((BN, 128 - NR), _f32)], axis=1)
    b_r[:, pl.ds(0, 128)] = _dot(h, wk_r[...])
    b_r[:, pl.ds(128, 128)] = _dot(h, wv_r[...])


def _init_call(x256, pos128, batchi, mol_x, wa, wb128, wc, b,
               wq, wk, wv, wekT):
    def body(x_r, p_r, bt_r, mx_r, wa_r, wb_r, wc_r, b_r,
             wq_r, wk_r, wv_r, we_r, o_r, a_r, bo_r):
        mw = _dot(mx_r[...], wc_r[...])
        gids = lax.broadcasted_iota(_i32, (1, G), 1)
        oh = (bt_r[...] == gids).astype(_f32)
        h = _dot(x_r[...], wa_r[...])
        h += _dot(p_r[...] * (1.0 / RMAX), wb_r[...])
        h += _dot(oh, mw)
        h = jax.nn.gelu(h + b_r[...])
        o_r[...] = h
        _emit_qkv(h, wq_r, wk_r, wv_r, we_r, a_r, bo_r)

    return pl.pallas_call(
        body,
        grid=(GN,),
        in_specs=[
            pl.BlockSpec((BN, 256), lambda i: (i, 0)),
            pl.BlockSpec((BN, 128), lambda i: (i, 0)),
            pl.BlockSpec((BN, 1), lambda i: (i, 0)),
            pl.BlockSpec((G, NR), lambda i: (0, 0)),
            pl.BlockSpec((256, HID), lambda i: (0, 0)),
            pl.BlockSpec((128, HID), lambda i: (0, 0)),
            pl.BlockSpec((NR, HID), lambda i: (0, 0)),
            pl.BlockSpec((1, HID), lambda i: (0, 0)),
            pl.BlockSpec((HID, HID), lambda i: (0, 0)),
            pl.BlockSpec((HID, HID), lambda i: (0, 0)),
            pl.BlockSpec((HID, HID), lambda i: (0, 0)),
            pl.BlockSpec((HID, NR), lambda i: (0, 0)),
        ],
        out_specs=[pl.BlockSpec((BN, HID), lambda i: (i, 0)),
                   pl.BlockSpec((BN, 256), lambda i: (i, 0)),
                   pl.BlockSpec((BN, 256), lambda i: (i, 0))],
        out_shape=[jax.ShapeDtypeStruct((NP, HID), _f32),
                   jax.ShapeDtypeStruct((NP, 256), _f32),
                   jax.ShapeDtypeStruct((NP, 256), _f32)],
    )(x256, pos128, batchi, mol_x, wa, wb128, wc, b, wq, wk, wv, wekT)


def _rbf_call(pd, ps):
    def body(pd_r, ps_r, r128_r):
        diff = pd_r[...] - ps_r[...] + 1e-8
        c128 = lax.broadcasted_iota(_i32, (BE, 128), 1)
        d2 = jnp.sum(jnp.where(c128 < 3, diff * diff, 0.0), axis=1,
                     keepdims=True)
        d = jnp.sqrt(d2)
        mu128 = c128.astype(_f32) * _MUSTEP
        vals = jnp.exp(-_GAMMA * (d - mu128) ** 2)
        r128_r[...] = jnp.where(c128 < NR, vals,
                                jnp.where(c128 == NR, 1.0, 0.0))

    return pl.pallas_call(
        body,
        grid=(GE,),
        in_specs=[pl.BlockSpec((BE, 128), lambda i: (i, 0)),
                  pl.BlockSpec((BE, 128), lambda i: (i, 0))],
        out_specs=pl.BlockSpec((BE, 128), lambda i: (i, 0)),
        out_shape=jax.ShapeDtypeStruct((EP, 128), _f32),
    )(pd, ps)


def _logits_call(qqed, kvs, rbf128):
    def body(qd_r, qe_r, ks_r, rb_r, l_r, m_r):
        i = pl.program_id(0)
        cols = lax.broadcasted_iota(_i32, (BE, 128), 1)
        qe = jnp.where(cols < NR, qe_r[...], 0.0)
        rb = jnp.where(cols < NR, rb_r[...], 0.0)
        raw = (jnp.sum(qd_r[...] * ks_r[...], axis=1, keepdims=True)
               + jnp.sum(qe * rb, axis=1, keepdims=True)) * _SQS
        eid = lax.broadcasted_iota(_i32, (BE, 1), 0) + i * BE
        lv = jnp.where(eid < E, raw, _NEG)
        l_r[...] = lv

        @pl.when(i == 0)
        def _():
            m_r[...] = jnp.full((1, 1), _NEG, _f32)

        m_r[...] = jnp.maximum(m_r[...], jnp.max(lv))

    return pl.pallas_call(
        body,
        grid=(GE,),
        in_specs=[pl.BlockSpec((BE, 128), lambda i: (i, 0)),
                  pl.BlockSpec((BE, 128), lambda i: (i, 1)),
                  pl.BlockSpec((BE, 128), lambda i: (i, 0)),
                  pl.BlockSpec((BE, 128), lambda i: (i, 0))],
        out_specs=[pl.BlockSpec((BE, 1), lambda i: (i, 0)),
                   pl.BlockSpec((1, 1), lambda i: (0, 0))],
        out_shape=[jax.ShapeDtypeStruct((EP, 1), _f32),
                   jax.ShapeDtypeStruct((1, 1), _f32)],
    )(qqed, qqed, kvs, rbf128)


def _payload_call(logits, m, kvs, rbf128):
    def body(l_r, m_r, vs_r, rb_r, p1_r, p2_r):
        a = jnp.exp(l_r[...] - m_r[...])
        p1_r[...] = a * vs_r[...]
        p2_r[...] = a * rb_r[...]

    return pl.pallas_call(
        body,
        grid=(GE,),
        in_specs=[pl.BlockSpec((BE, 1), lambda i: (i, 0)),
                  pl.BlockSpec((1, 1), lambda i: (0, 0)),
                  pl.BlockSpec((BE, 128), lambda i: (i, 1)),
                  pl.BlockSpec((BE, 128), lambda i: (i, 0))],
        out_specs=[pl.BlockSpec((BE, 128), lambda i: (i, 0)),
                   pl.BlockSpec((BE, 128), lambda i: (i, 0))],
        out_shape=[jax.ShapeDtypeStruct((EP, 128), _f32),
                   jax.ShapeDtypeStruct((EP, 128), _f32)],
    )(logits, m, kvs, rbf128)


def _epilogue_body(a_r, c_r, h_r, we_r, wr_r, g_r, be_r):
    acc1 = a_r[...]
    acc2 = c_r[...]
    rows = lax.broadcasted_iota(_i32, (128, 1), 0)
    cols = lax.broadcasted_iota(_i32, (128, NR), 1)
    ssel = (rows == cols).astype(_f32)
    dsel = (rows == NR).astype(_f32)
    s = _dot(acc2, ssel)
    denom = _dot(acc2, dsel)
    agg = (acc1 + _dot(s, we_r[...])) / (denom + 1e-16)
    out = agg + _dot(h_r[...], wr_r[...])
    mean = jnp.mean(out, axis=1, keepdims=True)
    cen = out - mean
    var = jnp.mean(cen * cen, axis=1, keepdims=True)
    hn = cen / jnp.sqrt(var + 1e-5)
    return jax.nn.gelu(hn * g_r[...] + be_r[...])


_EPI_SPECS = [pl.BlockSpec((BN, 128), lambda i: (i, 0)),
              pl.BlockSpec((BN, 128), lambda i: (i, 0)),
              pl.BlockSpec((BN, HID), lambda i: (i, 0)),
              pl.BlockSpec((NR, HID), lambda i: (0, 0)),
              pl.BlockSpec((HID, HID), lambda i: (0, 0)),
              pl.BlockSpec((1, HID), lambda i: (0, 0)),
              pl.BlockSpec((1, HID), lambda i: (0, 0))]


def _epilogue_call(o1, o2, h, wev, wr, lng, lnb):
    def body(a_r, c_r, h_r, we_r, wr_r, g_r, be_r, o_r):
        o_r[...] = _epilogue_body(a_r, c_r, h_r, we_r, wr_r, g_r, be_r)

    return pl.pallas_call(
        body,
        grid=(GN,),
        in_specs=_EPI_SPECS,
        out_specs=pl.BlockSpec((BN, HID), lambda i: (i, 0)),
        out_shape=jax.ShapeDtypeStruct((NP, HID), _f32),
    )(o1, o2, h, wev, wr, lng, lnb)


def _epilogue_qkv_call(o1, o2, h, wev, wr, lng, lnb, wq, wk, wv, wekT):
    def body(a_r, c_r, h_r, we_r, wr_r, g_r, be_r,
             wq_r, wk_r, wv_r, wek_r, o_r, ao_r, bo_r):
        hn = _epilogue_body(a_r, c_r, h_r, we_r, wr_r, g_r, be_r)
        o_r[...] = hn
        _emit_qkv(hn, wq_r, wk_r, wv_r, wek_r, ao_r, bo_r)

    return pl.pallas_call(
        body,
        grid=(GN,),
        in_specs=_EPI_SPECS + [pl.BlockSpec((HID, HID), lambda i: (0, 0)),
                               pl.BlockSpec((HID, HID), lambda i: (0, 0)),
                               pl.BlockSpec((HID, HID), lambda i: (0, 0)),
                               pl.BlockSpec((HID, NR), lambda i: (0, 0))],
        out_specs=[pl.BlockSpec((BN, HID), lambda i: (i, 0)),
                   pl.BlockSpec((BN, 256), lambda i: (i, 0)),
                   pl.BlockSpec((BN, 256), lambda i: (i, 0))],
        out_shape=[jax.ShapeDtypeStruct((NP, HID), _f32),
                   jax.ShapeDtypeStruct((NP, 256), _f32),
                   jax.ShapeDtypeStruct((NP, 256), _f32)],
    )(o1, o2, h, wev, wr, lng, lnb, wq, wk, wv, wekT)


def _emb_pool_call(h, batchi, wemb, bemb):
    def body(h_r, bt_r, w_r, b_r, o_r):
        i = pl.program_id(0)
        g = jax.nn.gelu(jnp.dot(h_r[...], w_r[...],
                                preferred_element_type=_f32,
                    precision=lax.Precision.HIGHEST) + b_r[...])

        @pl.when(i == 0)
        def _():
            o_r[...] = jnp.full((G, 2 * HID), _NEG, _f32)

        lo = bt_r[0, 0]
        hi = jnp.minimum(bt_r[BN - 1, 0], G - 1) + 1

        def gbody(gi, carry):
            mask = bt_r[...] == gi
            vals = jnp.where(mask, g, _NEG)
            m = jnp.max(vals, axis=0, keepdims=True)
            cur = o_r[pl.ds(gi, 1), :]
            o_r[pl.ds(gi, 1), :] = jnp.maximum(cur, m)
            return carry

        lax.fori_loop(lo, hi, gbody, 0)

    return pl.pallas_call(
        body,
        grid=(GN,),
        in_specs=[pl.BlockSpec((BN, HID), lambda i: (i, 0)),
                  pl.BlockSpec((BN, 1), lambda i: (i, 0)),
                  pl.BlockSpec((HID, 2 * HID), lambda i: (0, 0)),
                  pl.BlockSpec((1, 2 * HID), lambda i: (0, 0))],
        out_specs=pl.BlockSpec((G, 2 * HID), lambda i: (0, 0)),
        out_shape=jax.ShapeDtypeStruct((G, 2 * HID), _f32),
    )(h, batchi, wemb, bemb)


def _head_call(pooled, wfcs, bfcs, wout, bout):
    def body(p_r, w0, w1, w2, w3, b0, b1, b2, b3, wo, bo, o_r):
        f = p_r[...]
        f = jnp.where(f > -1e29, f, 0.0)
        for w_r, b_r in ((w0, b0), (w1, b1), (w2, b2), (w3, b3)):
            f = jax.nn.gelu(jnp.dot(f, w_r[...],
                                    preferred_element_type=_f32,
                    precision=lax.Precision.HIGHEST) + b_r[...])
        o_r[...] = jnp.dot(f, wo[...], preferred_element_type=_f32,
                    precision=lax.Precision.HIGHEST) + bo[...]

    emb = 2 * HID
    return pl.pallas_call(
        body,
        in_specs=[pl.BlockSpec((G, emb), lambda: (0, 0))]
        + [pl.BlockSpec((emb, emb), lambda: (0, 0))] * 4
        + [pl.BlockSpec((1, emb), lambda: (0, 0))] * 4
        + [pl.BlockSpec((emb, 12), lambda: (0, 0)),
           pl.BlockSpec((1, 12), lambda: (0, 0))],
        out_specs=pl.BlockSpec((G, 12), lambda: (0, 0)),
        out_shape=jax.ShapeDtypeStruct((G, 12), _f32),
    )(pooled, *wfcs, *bfcs, wout, bout)


def kernel(x, pos, edge_index, batch, mol_x, params):
    srci = jnp.zeros((EP,), _i32).at[:E].set(
        edge_index[0].astype(_i32)).reshape(EP // C, C)
    dsti = jnp.zeros((EP,), _i32).at[:E].set(
        edge_index[1].astype(_i32)).reshape(EP // C, C)
    pos128 = jnp.zeros((NP, 128), _f32).at[:N, :3].set(pos)
    x256 = jnp.zeros((NP, 256), _f32).at[:N].set(x[:, :256])
    batchi = jnp.full((NP, 1), G, _i32).at[:N, 0].set(batch.astype(_i32))

    wi = params['W_init']
    wa = wi[:256]
    wb128 = jnp.zeros((128, HID), _f32).at[:3].set(wi[256:259])
    wc = wi[259:291]
    b_init = params['b_init'].reshape(1, HID)

    h, qqe, kv = _init_call(x256, pos128, batchi, mol_x, wa, wb128, wc,
                            b_init, params['Wq0'], params['Wk0'],
                            params['Wv0'], params['Wek0'].T)
    pd, ps = _gather_pos_call(pos128, dsti, srci)
    rbf128 = _rbf_call(pd, ps)

    for l in range(NLAYERS):
        qqed, kvs = _gather_layer_call(qqe, kv, dsti, srci)
        logits, m = _logits_call(qqed, kvs, rbf128)
        p1, p2 = _payload_call(logits, m, kvs, rbf128)
        o1, o2 = _scatter_call(p1, p2, dsti)
        epi_args = (o1, o2, h, params['Wev%d' % l], params['Wr%d' % l],
                    params['lng%d' % l].reshape(1, HID),
                    params['lnb%d' % l].reshape(1, HID))
        if l < NLAYERS - 1:
            h, qqe, kv = _epilogue_qkv_call(
                *epi_args, params['Wq%d' % (l + 1)], params['Wk%d' % (l + 1)],
                params['Wv%d' % (l + 1)], params['Wek%d' % (l + 1)].T)
        else:
            h = _epilogue_call(*epi_args)

    pooled = _emb_pool_call(h, batchi, params['W_emb'],
                            params['b_emb'].reshape(1, 2 * HID))
    out = _head_call(pooled,
                     [params['Wfc%d' % l] for l in range(NFC)],
                     [params['bfc%d' % l].reshape(1, 2 * HID)
                      for l in range(NFC)],
                     params['W_out'], params['b_out'].reshape(1, 12))
    return out
